# baseline (speedup 1.0000x reference)
import functools

import jax
import jax.numpy as jnp
from jax import lax
from jax.experimental import pallas as pl
from jax.experimental.pallas import tpu as pltpu
from jax.experimental.pallas import tpu_sc as plsc

B, F, V, D = 4096, 26, 100000, 32
NW = 32
BPW = B // NW
EB = 4


def _tc_rowmajor(emb_t):
    CHUNKS = [(i * 12800, 12800) for i in range(7)] + [(89600, 10400)]

    def body(in_ref, out_ref, scr_ref, sem):
        f = pl.program_id(0)
        for off, sz in CHUNKS:
            scr_ref[pl.ds(0, sz), :] = in_ref[0, :, pl.ds(off, sz)].T
            pltpu.make_async_copy(
                scr_ref.at[pl.ds(0, sz), :],
                out_ref.at[f, pl.ds(off, sz), :], sem).start()
            pltpu.make_async_copy(
                scr_ref.at[pl.ds(0, sz), :],
                out_ref.at[f, pl.ds(off, sz), :], sem).wait()

    return pl.pallas_call(
        body,
        grid=(F,),
        in_specs=[pl.BlockSpec((1, D, V), lambda f: (f, 0, 0))],
        out_specs=pl.BlockSpec(memory_space=pl.ANY),
        out_shape=jax.ShapeDtypeStruct((F, V, D), jnp.float32),
        scratch_shapes=[
            pltpu.VMEM((12800, D), jnp.float32),
            pltpu.SemaphoreType.DMA,
        ],
    )(emb_t)


def _sc_emb_fm(vidx_r, emb_t):
    mesh = plsc.VectorSubcoreMesh(core_axis_name="c", subcore_axis_name="s")

    @functools.partial(
        pl.kernel,
        out_type=jax.ShapeDtypeStruct((B,), jnp.float32),
        mesh=mesh,
        scratch_types=[
            pltpu.VMEM((F, BPW), jnp.int32),
            pltpu.VMEM((F * EB, 8, D), jnp.float32),
            pltpu.VMEM((BPW * 16,), jnp.float32),
            pltpu.VMEM((BPW,), jnp.float32),
            pltpu.SemaphoreType.DMA,
        ],
        compiler_params=pltpu.CompilerParams(
            needs_layout_passes=False, use_tc_tiling_on_sc=True),
    )
    def body(vidx_hbm, emb_hbm, eout_hbm, idx_v, dst_v, tbuf_v, eout_v, sem):
        wid = lax.axis_index("c") * 16 + lax.axis_index("s")
        pltpu.sync_copy(vidx_hbm.at[wid], idx_v)

        def group_body(g, _):
            for sub in range(4):
                def fire_f(f, _, sub=sub):
                    va_vec = idx_v[f, pl.ds(g * 16, 16)] & ~7
                    for kk in range(EB):
                        va = pl.multiple_of(va_vec[sub * EB + kk], 8)
                        pltpu.async_copy(
                            emb_hbm.at[f, pl.ds(va, 8), :],
                            dst_v.at[f * EB + kk], sem)
                    return 0
                lax.fori_loop(0, F, fire_f, 0)

                def drain_f(f, _, sub=sub):
                    va_vec = idx_v[f, pl.ds(g * 16, 16)] & ~7
                    for kk in range(EB):
                        va = pl.multiple_of(va_vec[sub * EB + kk], 8)
                        pltpu.make_async_copy(
                            emb_hbm.at[f, pl.ds(va, 8), :],
                            dst_v.at[f * EB + kk], sem).wait()
                    return 0
                lax.fori_loop(0, F, drain_f, 0)

                for ll in range(EB):
                    def f_body(f, carry, sub=sub, ll=ll):
                        s0, s1, q0, q1 = carry
                        r = idx_v[f, pl.ds(g * 16, 16)][sub * EB + ll] & 7
                        v0 = dst_v[f * EB + ll, r, pl.ds(0, 16)]
                        v1 = dst_v[f * EB + ll, r, pl.ds(16, 16)]
                        return s0 + v0, s1 + v1, q0 + v0 * v0, q1 + v1 * v1
                    z = jnp.zeros((16,), jnp.float32)
                    s0, s1, q0, q1 = lax.fori_loop(0, F, f_body, (z, z, z, z))
                    tbuf_v[pl.ds((g * 16 + sub * EB + ll) * 16, 16)] = (
                        s0 * s0 + s1 * s1 - q0 - q1)
            return 0
        lax.fori_loop(0, BPW // 16, group_body, 0)

        idx16 = lax.iota(jnp.int32, 16)

        def eg_body(g, _):
            base = g * 256 + idx16 * 16
            def dd_body(dd, acc):
                return acc + plsc.load_gather(tbuf_v, [base + dd])
            acc = lax.fori_loop(0, 16, dd_body, jnp.zeros((16,), jnp.float32))
            eout_v[pl.ds(g * 16, 16)] = 0.5 * acc
            return 0
        lax.fori_loop(0, BPW // 16, eg_body, 0)

        pltpu.sync_copy(eout_v, eout_hbm.at[pl.ds(wid * BPW, BPW)])

    return body(vidx_r, emb_t)


def _sc_lin(idx_r, lin_flat):
    mesh = plsc.VectorSubcoreMesh(core_axis_name="c", subcore_axis_name="s")

    @functools.partial(
        pl.kernel,
        out_type=jax.ShapeDtypeStruct((B,), jnp.float32),
        mesh=mesh,
        scratch_types=[
            pltpu.VMEM((F, BPW), jnp.int32),
            pltpu.VMEM((F, BPW), jnp.float32),
            pltpu.VMEM((BPW,), jnp.float32),
            pltpu.SemaphoreType.DMA,
        ],
        compiler_params=pltpu.CompilerParams(
            needs_layout_passes=False, use_tc_tiling_on_sc=False),
    )
    def body(idx_hbm, lin_hbm, lout_hbm, idx_v, linv_v, lout_v, sem):
        wid = lax.axis_index("c") * 16 + lax.axis_index("s")
        pltpu.sync_copy(idx_hbm.at[wid], idx_v)

        def fire(f, _):
            pltpu.async_copy(lin_hbm.at[idx_v.at[f]], linv_v.at[f], sem)
            return 0
        lax.fori_loop(0, F, fire, 0)

        def drain(f, _):
            pltpu.make_async_copy(
                lin_hbm.at[idx_v.at[f]], linv_v.at[f], sem).wait()
            return 0
        lax.fori_loop(0, F, drain, 0)

        def g_body(g, _):
            def f_body(f, acc):
                return acc + linv_v[f, pl.ds(g * 16, 16)]
            acc = lax.fori_loop(0, F, f_body, jnp.zeros((16,), jnp.float32))
            lout_v[pl.ds(g * 16, 16)] = acc
            return 0
        lax.fori_loop(0, BPW // 16, g_body, 0)

        pltpu.sync_copy(lout_v, lout_hbm.at[pl.ds(wid * BPW, BPW)])

    return body(idx_r, lin_flat)


def _tc_broadcast(lin_col, emb_row, bias2):
    BR = 512

    def body(lin_ref, emb_ref, bias_ref, out_ref):
        out_ref[...] = lin_ref[...] + emb_ref[...] + bias_ref[0, 0]

    return pl.pallas_call(
        body,
        grid=(B // BR,),
        in_specs=[
            pl.BlockSpec((BR, 1), lambda i: (i, 0)),
            pl.BlockSpec((1, B), lambda i: (0, 0)),
            pl.BlockSpec(memory_space=pltpu.SMEM),
        ],
        out_specs=pl.BlockSpec((BR, B), lambda i: (i, 0)),
        out_shape=jax.ShapeDtypeStruct((B, B), jnp.float32),
    )(lin_col, emb_row, bias2)


def kernel(indices, emb_tables, lin_tables, bias):
    emb_rm = _tc_rowmajor(jnp.transpose(emb_tables, (0, 2, 1)))
    lin_flat = lin_tables.reshape(F * V)
    vidx_r = indices.reshape(NW, BPW, F).transpose(0, 2, 1)
    gidx = indices + (jnp.arange(F, dtype=jnp.int32) * V)[None, :]
    idx_r = gidx.reshape(NW, BPW, F).transpose(0, 2, 1)
    emb_logits = _sc_emb_fm(vidx_r, emb_rm)
    lin_sums = _sc_lin(idx_r, lin_flat)
    out = _tc_broadcast(lin_sums.reshape(B, 1), emb_logits.reshape(1, B),
                        bias.reshape(1, 1))
    return out


# speedup vs baseline: 2.3938x; 2.3938x over previous
import functools

import jax
import jax.numpy as jnp
from jax import lax
from jax.experimental import pallas as pl
from jax.experimental.pallas import tpu as pltpu
from jax.experimental.pallas import tpu_sc as plsc

B, F, V, D = 4096, 26, 100000, 32
NW = 32
BPW = B // NW
EB = 4


def _sc_emb_fm(vidx_r, emb_t):
    mesh = plsc.VectorSubcoreMesh(core_axis_name="c", subcore_axis_name="s")

    @functools.partial(
        pl.kernel,
        out_type=jax.ShapeDtypeStruct((B,), jnp.float32),
        mesh=mesh,
        scratch_types=[
            pltpu.VMEM((F, BPW), jnp.int32),
            pltpu.VMEM((F * EB, 8, D), jnp.float32),
            pltpu.VMEM((BPW * 16,), jnp.float32),
            pltpu.VMEM((BPW,), jnp.float32),
            pltpu.SemaphoreType.DMA,
        ],
        compiler_params=pltpu.CompilerParams(
            needs_layout_passes=False, use_tc_tiling_on_sc=True),
    )
    def body(vidx_hbm, emb_hbm, eout_hbm, idx_v, dst_v, tbuf_v, eout_v, sem):
        wid = lax.axis_index("c") * 16 + lax.axis_index("s")
        pltpu.sync_copy(vidx_hbm.at[wid], idx_v)

        def group_body(g, _):
            for sub in range(4):
                def fire_f(f, _, sub=sub):
                    va_vec = idx_v[f, pl.ds(g * 16, 16)] & ~7
                    for kk in range(EB):
                        va = pl.multiple_of(va_vec[sub * EB + kk], 8)
                        pltpu.async_copy(
                            emb_hbm.at[pl.ds(va, 8), :],
                            dst_v.at[f * EB + kk], sem)
                    return 0
                lax.fori_loop(0, F, fire_f, 0)

                def drain_f(f, _, sub=sub):
                    va_vec = idx_v[f, pl.ds(g * 16, 16)] & ~7
                    for kk in range(EB):
                        va = pl.multiple_of(va_vec[sub * EB + kk], 8)
                        pltpu.make_async_copy(
                            emb_hbm.at[pl.ds(va, 8), :],
                            dst_v.at[f * EB + kk], sem).wait()
                    return 0
                lax.fori_loop(0, F, drain_f, 0)

                for ll in range(EB):
                    def f_body(f, carry, sub=sub, ll=ll):
                        s0, s1, q0, q1 = carry
                        r = idx_v[f, pl.ds(g * 16, 16)][sub * EB + ll] & 7
                        v0 = dst_v[f * EB + ll, r, pl.ds(0, 16)]
                        v1 = dst_v[f * EB + ll, r, pl.ds(16, 16)]
                        return s0 + v0, s1 + v1, q0 + v0 * v0, q1 + v1 * v1
                    z = jnp.zeros((16,), jnp.float32)
                    s0, s1, q0, q1 = lax.fori_loop(0, F, f_body, (z, z, z, z))
                    tbuf_v[pl.ds((g * 16 + sub * EB + ll) * 16, 16)] = (
                        s0 * s0 + s1 * s1 - q0 - q1)
            return 0
        lax.fori_loop(0, BPW // 16, group_body, 0)

        idx16 = lax.iota(jnp.int32, 16)

        def eg_body(g, _):
            base = g * 256 + idx16 * 16
            def dd_body(dd, acc):
                return acc + plsc.load_gather(tbuf_v, [base + dd])
            acc = lax.fori_loop(0, 16, dd_body, jnp.zeros((16,), jnp.float32))
            eout_v[pl.ds(g * 16, 16)] = 0.5 * acc
            return 0
        lax.fori_loop(0, BPW // 16, eg_body, 0)

        pltpu.sync_copy(eout_v, eout_hbm.at[pl.ds(wid * BPW, BPW)])

    return body(vidx_r, emb_t)


def _sc_lin(idx_r, lin_flat):
    mesh = plsc.VectorSubcoreMesh(core_axis_name="c", subcore_axis_name="s")

    @functools.partial(
        pl.kernel,
        out_type=jax.ShapeDtypeStruct((B,), jnp.float32),
        mesh=mesh,
        scratch_types=[
            pltpu.VMEM((F, BPW), jnp.int32),
            pltpu.VMEM((F, BPW), jnp.float32),
            pltpu.VMEM((BPW,), jnp.float32),
            pltpu.SemaphoreType.DMA,
        ],
        compiler_params=pltpu.CompilerParams(
            needs_layout_passes=False, use_tc_tiling_on_sc=False),
    )
    def body(idx_hbm, lin_hbm, lout_hbm, idx_v, linv_v, lout_v, sem):
        wid = lax.axis_index("c") * 16 + lax.axis_index("s")
        pltpu.sync_copy(idx_hbm.at[wid], idx_v)

        def fire(f, _):
            pltpu.async_copy(lin_hbm.at[idx_v.at[f]], linv_v.at[f], sem)
            return 0
        lax.fori_loop(0, F, fire, 0)

        def drain(f, _):
            pltpu.make_async_copy(
                lin_hbm.at[idx_v.at[f]], linv_v.at[f], sem).wait()
            return 0
        lax.fori_loop(0, F, drain, 0)

        def g_body(g, _):
            def f_body(f, acc):
                return acc + linv_v[f, pl.ds(g * 16, 16)]
            acc = lax.fori_loop(0, F, f_body, jnp.zeros((16,), jnp.float32))
            lout_v[pl.ds(g * 16, 16)] = acc
            return 0
        lax.fori_loop(0, BPW // 16, g_body, 0)

        pltpu.sync_copy(lout_v, lout_hbm.at[pl.ds(wid * BPW, BPW)])

    return body(idx_r, lin_flat)


def _tc_broadcast(lin_col, emb_row, bias2):
    BR = 512

    def body(lin_ref, emb_ref, bias_ref, out_ref):
        out_ref[...] = lin_ref[...] + emb_ref[...] + bias_ref[0, 0]

    return pl.pallas_call(
        body,
        grid=(B // BR,),
        in_specs=[
            pl.BlockSpec((BR, 1), lambda i: (i, 0)),
            pl.BlockSpec((1, B), lambda i: (0, 0)),
            pl.BlockSpec(memory_space=pltpu.SMEM),
        ],
        out_specs=pl.BlockSpec((BR, B), lambda i: (i, 0)),
        out_shape=jax.ShapeDtypeStruct((B, B), jnp.float32),
    )(lin_col, emb_row, bias2)


def kernel(indices, emb_tables, lin_tables, bias):
    emb_rm = emb_tables.reshape(F * V, D)
    lin_flat = lin_tables.reshape(F * V)
    gidx = indices + (jnp.arange(F, dtype=jnp.int32) * V)[None, :]
    idx_r = gidx.reshape(NW, BPW, F).transpose(0, 2, 1)
    emb_logits = _sc_emb_fm(idx_r, emb_rm)
    lin_sums = _sc_lin(idx_r, lin_flat)
    out = _tc_broadcast(lin_sums.reshape(B, 1), emb_logits.reshape(1, B),
                        bias.reshape(1, 1))
    return out


# speedup vs baseline: 3.6006x; 1.5042x over previous
import functools

import jax
import jax.numpy as jnp
from jax import lax
from jax.experimental import pallas as pl
from jax.experimental.pallas import tpu as pltpu
from jax.experimental.pallas import tpu_sc as plsc

B, F, V, D = 4096, 26, 100000, 32
NW = 32
BPW = B // NW
EB = 2


def _sc_emb_fm(vidx_r, emb_t):
    mesh = plsc.VectorSubcoreMesh(core_axis_name="c", subcore_axis_name="s")

    @functools.partial(
        pl.kernel,
        out_type=jax.ShapeDtypeStruct((B,), jnp.float32),
        mesh=mesh,
        scratch_types=[
            pltpu.VMEM((F, BPW), jnp.int32),
            pltpu.VMEM((2, F * EB, 8, D), jnp.float32),
            pltpu.VMEM((BPW * 16,), jnp.float32),
            pltpu.VMEM((BPW,), jnp.float32),
            pltpu.SemaphoreType.DMA,
            pltpu.SemaphoreType.DMA,
        ],
        compiler_params=pltpu.CompilerParams(
            needs_layout_passes=False, use_tc_tiling_on_sc=True),
    )
    def body(vidx_hbm, emb_hbm, eout_hbm, idx_v, dst_v, tbuf_v, eout_v,
             sem0, sem1):
        wid = lax.axis_index("c") * 16 + lax.axis_index("s")
        pltpu.sync_copy(vidx_hbm.at[wid], idx_v)

        def group_body(g, _):
            sems = (sem0, sem1)

            def fire(sub, buf):
                sem = sems[buf]
                def fire_f(f, _):
                    va_vec = idx_v[f, pl.ds(g * 16, 16)] & ~7
                    for kk in range(EB):
                        va = pl.multiple_of(va_vec[sub * EB + kk], 8)
                        pltpu.async_copy(
                            emb_hbm.at[pl.ds(va, 8), :],
                            dst_v.at[buf, f * EB + kk], sem)
                    return 0
                lax.fori_loop(0, F, fire_f, 0)

            def drain(sub, buf):
                sem = sems[buf]
                def drain_f(f, _):
                    va_vec = idx_v[f, pl.ds(g * 16, 16)] & ~7
                    for kk in range(EB):
                        va = pl.multiple_of(va_vec[sub * EB + kk], 8)
                        pltpu.make_async_copy(
                            emb_hbm.at[pl.ds(va, 8), :],
                            dst_v.at[buf, f * EB + kk], sem).wait()
                    return 0
                lax.fori_loop(0, F, drain_f, 0)

            fire(0, 0)
            for sub in range(16 // EB):
                buf = sub % 2
                if sub + 1 < 16 // EB:
                    fire(sub + 1, 1 - buf)
                drain(sub, buf)
                for ll in range(EB):
                    def f_body(f, carry, sub=sub, ll=ll, buf=buf):
                        s0, s1, q0, q1 = carry
                        r = idx_v[f, pl.ds(g * 16, 16)][sub * EB + ll] & 7
                        v0 = dst_v[buf, f * EB + ll, r, pl.ds(0, 16)]
                        v1 = dst_v[buf, f * EB + ll, r, pl.ds(16, 16)]
                        return s0 + v0, s1 + v1, q0 + v0 * v0, q1 + v1 * v1
                    z = jnp.zeros((16,), jnp.float32)
                    s0, s1, q0, q1 = lax.fori_loop(0, F, f_body, (z, z, z, z))
                    tbuf_v[pl.ds((g * 16 + sub * EB + ll) * 16, 16)] = (
                        s0 * s0 + s1 * s1 - q0 - q1)
            return 0
        idx16 = lax.iota(jnp.int32, 16)

        def eg_body(g, _):
            base = g * 256 + idx16 * 16
            def dd_body(dd, acc):
                return acc + plsc.load_gather(tbuf_v, [base + dd])
            acc = lax.fori_loop(0, 16, dd_body, jnp.zeros((16,), jnp.float32))
            eout_v[pl.ds(g * 16, 16)] = 0.5 * acc
            return 0
        lax.fori_loop(0, BPW // 16, eg_body, 0)

        pltpu.sync_copy(eout_v, eout_hbm.at[pl.ds(wid * BPW, BPW)])

    return body(vidx_r, emb_t)


def _sc_lin(idx_r, lin_flat):
    mesh = plsc.VectorSubcoreMesh(core_axis_name="c", subcore_axis_name="s")

    @functools.partial(
        pl.kernel,
        out_type=jax.ShapeDtypeStruct((B,), jnp.float32),
        mesh=mesh,
        scratch_types=[
            pltpu.VMEM((F, BPW), jnp.int32),
            pltpu.VMEM((F, BPW), jnp.float32),
            pltpu.VMEM((BPW,), jnp.float32),
            pltpu.SemaphoreType.DMA,
        ],
        compiler_params=pltpu.CompilerParams(
            needs_layout_passes=False, use_tc_tiling_on_sc=False),
    )
    def body(idx_hbm, lin_hbm, lout_hbm, idx_v, linv_v, lout_v, sem):
        wid = lax.axis_index("c") * 16 + lax.axis_index("s")
        pltpu.sync_copy(idx_hbm.at[wid], idx_v)

        def fire(f, _):
            pltpu.async_copy(lin_hbm.at[idx_v.at[f]], linv_v.at[f], sem)
            return 0
        lax.fori_loop(0, F, fire, 0)

        def drain(f, _):
            pltpu.make_async_copy(
                lin_hbm.at[idx_v.at[f]], linv_v.at[f], sem).wait()
            return 0
        lax.fori_loop(0, F, drain, 0)

        def g_body(g, _):
            def f_body(f, acc):
                return acc + linv_v[f, pl.ds(g * 16, 16)]
            acc = lax.fori_loop(0, F, f_body, jnp.zeros((16,), jnp.float32))
            lout_v[pl.ds(g * 16, 16)] = acc
            return 0
        lax.fori_loop(0, BPW // 16, g_body, 0)

        pltpu.sync_copy(lout_v, lout_hbm.at[pl.ds(wid * BPW, BPW)])

    return body(idx_r, lin_flat)


def _tc_broadcast(lin_col, emb_row, bias2):
    BR = 512

    def body(lin_ref, emb_ref, bias_ref, out_ref):
        out_ref[...] = lin_ref[...] + emb_ref[...] + bias_ref[0, 0]

    return pl.pallas_call(
        body,
        grid=(B // BR,),
        in_specs=[
            pl.BlockSpec((BR, 1), lambda i: (i, 0)),
            pl.BlockSpec((1, B), lambda i: (0, 0)),
            pl.BlockSpec(memory_space=pltpu.SMEM),
        ],
        out_specs=pl.BlockSpec((BR, B), lambda i: (i, 0)),
        out_shape=jax.ShapeDtypeStruct((B, B), jnp.float32),
    )(lin_col, emb_row, bias2)


def kernel(indices, emb_tables, lin_tables, bias):
    emb_rm = emb_tables.reshape(F * V, D)
    lin_flat = lin_tables.reshape(F * V)
    gidx = indices + (jnp.arange(F, dtype=jnp.int32) * V)[None, :]
    idx_r = gidx.reshape(NW, BPW, F).transpose(0, 2, 1)
    emb_logits = _sc_emb_fm(idx_r, emb_rm)
    lin_sums = _sc_lin(idx_r, lin_flat)
    out = _tc_broadcast(lin_sums.reshape(B, 1), emb_logits.reshape(1, B),
                        bias.reshape(1, 1))
    return out
